# baseline (speedup 1.0000x reference)
import functools
import math

import jax
import jax.numpy as jnp
from jax import lax
from jax.experimental import pallas as pl
from jax.experimental.pallas import tpu as pltpu
from jax.experimental.pallas import tpu_sc as plsc

_LANES = 16
_NBUF = 8
_LEAD = _NBUF - 1
_HPB = 2


def _build(batch, seq, dim, vocab):
    nc, ns = 2, 16
    nw = nc * ns
    spw = seq // nw
    scale = math.sqrt(dim)
    mesh = plsc.VectorSubcoreMesh(
        core_axis_name="c", subcore_axis_name="s", num_cores=nc, num_subcores=ns
    )

    rpb = seq // 128
    rb = spw // _HPB
    scratch = [
        pltpu.VMEM((batch,), jnp.int32),
        pltpu.VMEM((batch, 128), jnp.int32),
        pltpu.VMEM((spw, dim), jnp.float32),
    ]
    scratch += [pltpu.VMEM((rb, dim), jnp.float32) for _ in range(_NBUF)]
    scratch += [pltpu.SemaphoreType.DMA for _ in range(2 * _NBUF)]

    @functools.partial(
        pl.kernel,
        out_type=jax.ShapeDtypeStruct((batch, seq, dim), jnp.float32),
        mesh=mesh,
        scratch_types=scratch,
    )
    def emb(idx_hbm, pos_hbm, table_hbm, out_hbm, rid_v, idx_v, pos_v,
            *bufs_sems):
        bufs = bufs_sems[:_NBUF]
        gsem = bufs_sems[_NBUF:2 * _NBUF]
        ssem = bufs_sems[2 * _NBUF:]

        w = lax.axis_index("s") * nc + lax.axis_index("c")
        s0 = w * spw

        q = lax.rem(w, 128 // spw)
        row0 = lax.div(w, 128 // spw)
        for k in range(batch // _LANES):
            rid_v[pl.ds(k * _LANES, _LANES)] = (
                lax.iota(jnp.int32, _LANES) * rpb
                + (row0 + k * _LANES * rpb)
            )
        pltpu.sync_copy(idx_hbm.at[rid_v], idx_v)
        pltpu.sync_copy(pos_hbm.at[pl.ds(s0, spw)], pos_v)

        nblk = batch * _HPB

        def gather(h, t):
            b = lax.div(h, _HPB)
            off = lax.rem(h, _HPB) * rb
            pltpu.async_copy(
                table_hbm.at[idx_v.at[b, pl.ds(q * spw + off, rb)]],
                bufs[t], gsem[t])

        def wait_gather(t):
            pltpu.make_async_copy(
                table_hbm.at[idx_v.at[0, pl.ds(0, rb)]], bufs[t],
                gsem[t]).wait()

        def store(h, t):
            b = lax.div(h, _HPB)
            off = lax.rem(h, _HPB) * rb
            pltpu.async_copy(bufs[t], out_hbm.at[b, pl.ds(s0 + off, rb), :],
                             ssem[t])

        def wait_store(t):
            pltpu.make_async_copy(bufs[t], out_hbm.at[0, pl.ds(s0, rb), :],
                                  ssem[t]).wait()

        def compute(h, t):
            buf = bufs[t]
            off = lax.rem(h, _HPB) * rb

            @plsc.parallel_loop(0, rb)
            def row(i):
                for j in range(dim // _LANES):
                    sl = pl.ds(j * _LANES, _LANES)
                    buf[i, sl] = buf[i, sl] * scale + pos_v[off + i, sl]

        for t in range(_NBUF):
            gather(jnp.int32(t), t)

        @pl.loop(0, nblk // _NBUF)
        def _block(g):
            for t in range(_NBUF):
                h = g * _NBUF + t
                wait_gather(t)
                compute(h, t)
                store(h, t)
                nh = h + _LEAD
                tn = (t + _LEAD) % _NBUF

                @pl.when(jnp.logical_and(nh >= _NBUF, nh < nblk))
                def _():
                    wait_store(tn)
                    gather(nh, tn)

        for t in range(_NBUF):
            wait_store(t)

    return emb


def kernel(tokens, token_table, pos_table):
    batch, seq = tokens.shape
    vocab, dim = token_table.shape
    emb = _build(batch, seq, dim, vocab)
    idx = tokens.astype(jnp.int32).reshape(batch * seq // 128, 128)
    return emb(idx, pos_table[:seq], token_table)


# speedup vs baseline: 1.0467x; 1.0467x over previous
import functools
import math

import jax
import jax.numpy as jnp
from jax import lax
from jax.experimental import pallas as pl
from jax.experimental.pallas import tpu as pltpu
from jax.experimental.pallas import tpu_sc as plsc

_LANES = 16
_NBUF = 3
_LEAD = _NBUF - 1
_BPB = 2


def _build(batch, seq, dim, vocab):
    nc, ns = 2, 16
    nw = nc * ns
    spw = seq // nw
    rb = _BPB * spw
    nblk = batch // _BPB
    scale = math.sqrt(dim)
    mesh = plsc.VectorSubcoreMesh(
        core_axis_name="c", subcore_axis_name="s", num_cores=nc, num_subcores=ns
    )

    scratch = [
        pltpu.VMEM((nblk, rb), jnp.int32),
        pltpu.VMEM((spw, dim), jnp.float32),
    ]
    scratch += [pltpu.VMEM((rb, dim), jnp.float32) for _ in range(_NBUF)]
    scratch += [pltpu.SemaphoreType.DMA for _ in range(2 * _NBUF)]

    @functools.partial(
        pl.kernel,
        out_type=jax.ShapeDtypeStruct((batch, seq, dim), jnp.float32),
        mesh=mesh,
        scratch_types=scratch,
    )
    def emb(idx_hbm, pos_hbm, table_hbm, out_hbm, idx_v, pos_v, *bufs_sems):
        bufs = bufs_sems[:_NBUF]
        gsem = bufs_sems[_NBUF:2 * _NBUF]
        ssem = bufs_sems[2 * _NBUF:]

        w = lax.axis_index("s") * nc + lax.axis_index("c")
        s0 = w * spw

        pltpu.sync_copy(idx_hbm.at[w], idx_v)
        pltpu.sync_copy(pos_hbm.at[pl.ds(s0, spw)], pos_v)

        def gather(h, t):
            pltpu.async_copy(table_hbm.at[idx_v.at[h]], bufs[t], gsem[t])

        def wait_gather(t):
            pltpu.make_async_copy(table_hbm.at[idx_v.at[0]], bufs[t],
                                  gsem[t]).wait()

        def store(h, t):
            for p in range(_BPB):
                pltpu.async_copy(
                    bufs[t].at[pl.ds(p * spw, spw)],
                    out_hbm.at[h * _BPB + p, pl.ds(s0, spw), :], ssem[t])

        def wait_store(t):
            for p in range(_BPB):
                pltpu.make_async_copy(
                    bufs[t].at[pl.ds(p * spw, spw)],
                    out_hbm.at[0, pl.ds(s0, spw), :], ssem[t]).wait()

        def compute(t):
            buf = bufs[t]
            for p in range(_BPB):

                @plsc.parallel_loop(0, spw)
                def row(i):
                    for j in range(dim // _LANES):
                        sl = pl.ds(j * _LANES, _LANES)
                        buf[p * spw + i, sl] = (
                            buf[p * spw + i, sl] * scale + pos_v[i, sl])

        def step(h, t):
            wait_gather(t)
            compute(t)
            store(h, t)
            nh = h + _LEAD
            tn = (t + _LEAD) % _NBUF

            @pl.when(jnp.logical_and(nh >= _NBUF, nh < nblk))
            def _():
                wait_store(tn)
                gather(nh, tn)

        for t in range(_NBUF):
            gather(jnp.int32(t), t)

        main = (nblk // _NBUF) * _NBUF

        @pl.loop(0, main // _NBUF)
        def _block(g):
            for t in range(_NBUF):
                step(g * _NBUF + t, t)

        for h in range(main, nblk):
            step(jnp.int32(h), h % _NBUF)

        for t in range(_NBUF):
            wait_store(t)

    return emb


def kernel(tokens, token_table, pos_table):
    batch, seq = tokens.shape
    vocab, dim = token_table.shape
    nw = 32
    spw = seq // nw
    idx = (tokens.astype(jnp.int32)
           .reshape(batch, nw, spw)
           .transpose(1, 0, 2)
           .reshape(nw, batch // _BPB, _BPB * spw))
    emb = _build(batch, seq, dim, vocab)
    return emb(idx, pos_table[:seq], token_table)


# speedup vs baseline: 1.1013x; 1.0522x over previous
import functools
import math

import jax
import jax.numpy as jnp
from jax import lax
from jax.experimental import pallas as pl
from jax.experimental.pallas import tpu as pltpu
from jax.experimental.pallas import tpu_sc as plsc

_LANES = 16
_NBUF = 4
_LEAD = _NBUF - 1


def _build(batch, seq, dim, vocab):
    nc, ns = 2, 16
    nw = nc * ns
    spw = seq // nw
    scale = math.sqrt(dim)
    mesh = plsc.VectorSubcoreMesh(
        core_axis_name="c", subcore_axis_name="s", num_cores=nc, num_subcores=ns
    )

    scratch = [
        pltpu.VMEM((batch, spw), jnp.int32),
        pltpu.VMEM((spw, dim), jnp.float32),
    ]
    scratch += [pltpu.VMEM((spw, dim), jnp.float32) for _ in range(_NBUF)]
    scratch += [pltpu.SemaphoreType.DMA for _ in range(2 * _NBUF)]

    @functools.partial(
        pl.kernel,
        out_type=jax.ShapeDtypeStruct((batch, seq, dim), jnp.float32),
        mesh=mesh,
        scratch_types=scratch,
    )
    def emb(idx_hbm, pos_hbm, table_hbm, out_hbm, idx_v, pos_v, *bufs_sems):
        bufs = bufs_sems[:_NBUF]
        gsem = bufs_sems[_NBUF:2 * _NBUF]
        ssem = bufs_sems[2 * _NBUF:]

        w = lax.axis_index("s") * nc + lax.axis_index("c")
        s0 = w * spw

        pltpu.sync_copy(idx_hbm.at[w], idx_v)
        pltpu.sync_copy(pos_hbm.at[pl.ds(s0, spw)], pos_v)

        def gather(b, t):
            pltpu.async_copy(table_hbm.at[idx_v.at[b]], bufs[t], gsem[t])

        def wait_gather(t):
            pltpu.make_async_copy(table_hbm.at[idx_v.at[0]], bufs[t],
                                  gsem[t]).wait()

        def store(b, t):
            pltpu.async_copy(bufs[t], out_hbm.at[b, pl.ds(s0, spw), :],
                             ssem[t])

        def wait_store(t):
            pltpu.make_async_copy(bufs[t], out_hbm.at[0, pl.ds(s0, spw), :],
                                  ssem[t]).wait()

        def compute(t):
            buf = bufs[t]

            def row(i, _):
                for j in range(dim // _LANES):
                    sl = pl.ds(j * _LANES, _LANES)
                    buf[i, sl] = buf[i, sl] * scale + pos_v[i, sl]
                return ()

            lax.fori_loop(0, spw, row, ())

        for t in range(_NBUF):
            gather(t, t)

        @pl.loop(0, batch // _NBUF)
        def _block(g):
            for t in range(_NBUF):
                b = g * _NBUF + t
                wait_gather(t)
                compute(t)
                store(b, t)
                nb = b + _LEAD
                tn = (t + _LEAD) % _NBUF

                @pl.when(jnp.logical_and(nb >= _NBUF, nb < batch))
                def _():
                    wait_store(tn)
                    gather(nb, tn)

        for t in range(_NBUF):
            wait_store(t)

    return emb


def kernel(tokens, token_table, pos_table):
    batch, seq = tokens.shape
    vocab, dim = token_table.shape
    nw = 32
    spw = seq // nw
    idx = tokens.astype(jnp.int32).reshape(batch, nw, spw).transpose(1, 0, 2)
    emb = _build(batch, seq, dim, vocab)
    return emb(idx, pos_table[:seq], token_table)


# speedup vs baseline: 1.1063x; 1.0045x over previous
import functools
import math

import jax
import jax.numpy as jnp
from jax import lax
from jax.experimental import pallas as pl
from jax.experimental.pallas import tpu as pltpu
from jax.experimental.pallas import tpu_sc as plsc

_LANES = 16
_NBUF = 4
_LEAD = _NBUF - 1


def _build(batch, seq, dim, vocab):
    nc, ns = 2, 16
    nw = nc * ns
    spw = seq // nw
    scale = math.sqrt(dim)
    mesh = plsc.VectorSubcoreMesh(
        core_axis_name="c", subcore_axis_name="s", num_cores=nc, num_subcores=ns
    )

    scratch = [
        pltpu.VMEM((batch, spw), jnp.int32),
        pltpu.VMEM((spw, dim), jnp.float32),
    ]
    scratch += [pltpu.VMEM((spw, dim), jnp.float32) for _ in range(_NBUF)]
    scratch += [pltpu.SemaphoreType.DMA for _ in range(2 * _NBUF)]

    @functools.partial(
        pl.kernel,
        out_type=jax.ShapeDtypeStruct((batch, seq, dim), jnp.float32),
        mesh=mesh,
        scratch_types=scratch,
    )
    def emb(idx_hbm, pos_hbm, table_hbm, out_hbm, idx_v, pos_v, *bufs_sems):
        bufs = bufs_sems[:_NBUF]
        gsem = bufs_sems[_NBUF:2 * _NBUF]
        ssem = bufs_sems[2 * _NBUF:]

        w = lax.axis_index("s") * nc + lax.axis_index("c")
        s0 = w * spw

        pltpu.sync_copy(idx_hbm.at[w], idx_v)

        def gather(b, t):
            pltpu.async_copy(table_hbm.at[idx_v.at[b]], bufs[t], gsem[t])

        def wait_gather(t):
            pltpu.make_async_copy(table_hbm.at[idx_v.at[0]], bufs[t],
                                  gsem[t]).wait()

        def store(b, t):
            pltpu.async_copy(bufs[t], out_hbm.at[b, pl.ds(s0, spw), :],
                             ssem[t])

        def wait_store(t):
            pltpu.make_async_copy(bufs[t], out_hbm.at[0, pl.ds(s0, spw), :],
                                  ssem[t]).wait()

        def compute(t):
            buf = bufs[t]

            def row(i, _):
                for j in range(dim // _LANES):
                    sl = pl.ds(j * _LANES, _LANES)
                    buf[i, sl] = buf[i, sl] * scale + pos_v[i, sl]
                return ()

            lax.fori_loop(0, spw, row, ())

        for t in range(_NBUF):
            gather(t, t)
        pltpu.sync_copy(pos_hbm.at[pl.ds(s0, spw)], pos_v)

        @pl.loop(0, batch // _NBUF)
        def _block(g):
            for t in range(_NBUF):
                b = g * _NBUF + t
                wait_gather(t)
                compute(t)
                store(b, t)
                nb = b + _LEAD
                tn = (t + _LEAD) % _NBUF

                @pl.when(jnp.logical_and(nb >= _NBUF, nb < batch))
                def _():
                    wait_store(tn)
                    gather(nb, tn)

        for t in range(_NBUF):
            wait_store(t)

    return emb


def kernel(tokens, token_table, pos_table):
    batch, seq = tokens.shape
    vocab, dim = token_table.shape
    nw = 32
    spw = seq // nw
    idx = tokens.astype(jnp.int32).reshape(batch, nw, spw).transpose(1, 0, 2)
    emb = _build(batch, seq, dim, vocab)
    return emb(idx, pos_table[:seq], token_table)
